# baseline (speedup 1.0000x reference)
import jax
import jax.numpy as jnp
from jax.experimental import pallas as pl
from jax.experimental.pallas import tpu as pltpu

TOKENS = 32768
HIDDEN = 4096
EXPERTS = 64
BLOCK_T = 1024
XBUFS = 3


def _inner(x_ref, o_ref, w_ref, b_ref):
    logits = jax.lax.dot_general(
        x_ref[...], w_ref[...],
        dimension_numbers=(((1,), (1,)), ((), ())),
        preferred_element_type=jnp.float32,
    )
    logits = logits + b_ref[...]
    m = jnp.max(logits, axis=-1, keepdims=True)
    e = jnp.exp(logits - m)
    o_ref[...] = e / jnp.sum(e, axis=-1, keepdims=True)


def _outer(x_hbm, w_ref, b_ref, o_hbm):
    w = w_ref
    bias = b_ref

    def body(x_ref, o_ref):
        _inner(x_ref, o_ref, w, bias)

    pipeline = pltpu.emit_pipeline(
        body,
        grid=(TOKENS // BLOCK_T,),
        in_specs=[
            pl.BlockSpec((BLOCK_T, HIDDEN), lambda i: (i, 0),
                         pipeline_mode=pl.Buffered(buffer_count=XBUFS)),
        ],
        out_specs=[
            pl.BlockSpec((BLOCK_T, EXPERTS), lambda i: (i, 0)),
        ],
    )
    pipeline(x_hbm, o_hbm)


def kernel(x, W, b):
    b2 = b.reshape(1, EXPERTS)
    return pl.pallas_call(
        _outer,
        in_specs=[
            pl.BlockSpec(memory_space=pltpu.MemorySpace.HBM),
            pl.BlockSpec((EXPERTS, HIDDEN), lambda: (0, 0)),
            pl.BlockSpec((1, EXPERTS), lambda: (0, 0)),
        ],
        out_specs=pl.BlockSpec(memory_space=pltpu.MemorySpace.HBM),
        out_shape=jax.ShapeDtypeStruct((TOKENS, EXPERTS), jnp.float32),
    )(x, W, b2)
